# baseline (speedup 1.0000x reference)
import functools

import jax
import jax.numpy as jnp
from jax import lax
from jax.experimental import pallas as pl
from jax.experimental.pallas import tpu as pltpu
from jax.experimental.pallas import tpu_sc as plsc

NE = 320000
D = 128
W = D // 2
NC, NS, L = 2, 16, 16
NW = NC * NS
E_PER_W = NE // NW
CHUNK = 80
NCHUNK = E_PER_W // CHUNK

_mesh = plsc.VectorSubcoreMesh(core_axis_name="c", subcore_axis_name="s")


def _lane_take(x, idx):
    dnums = lax.GatherDimensionNumbers(
        offset_dims=(), collapsed_slice_dims=(0,), start_index_map=(0,))
    return lax.gather(x, idx[:, None], dnums, slice_sizes=(1,),
                      mode=lax.GatherScatterMode.PROMISE_IN_BOUNDS)


@functools.partial(
    pl.kernel,
    mesh=_mesh,
    out_type=jax.ShapeDtypeStruct((NE,), jnp.float32),
    compiler_params=pltpu.CompilerParams(needs_layout_passes=False,
                                         use_tc_tiling_on_sc=False),
    scratch_types=[
        pltpu.VMEM_SHARED((10000, W), jnp.float32),
        pltpu.VMEM_SHARED((10000, W), jnp.float32),
        pltpu.VMEM((E_PER_W,), jnp.int32),
        pltpu.VMEM((E_PER_W,), jnp.int32),
        pltpu.VMEM((CHUNK, W), jnp.float32),
        pltpu.VMEM((CHUNK, W), jnp.float32),
        pltpu.VMEM((CHUNK, W), jnp.float32),
        pltpu.VMEM((CHUNK, W), jnp.float32),
        pltpu.VMEM((CHUNK,), jnp.float32),
        pltpu.SemaphoreType.DMA,
        pltpu.SemaphoreType.DMA,
        pltpu.SemaphoreType.DMA,
        pltpu.SemaphoreType.DMA,
    ],
)
def _sc_dot_kernel(xp_hbm, xd_hbm, idx_hbm, out_hbm,
                   sh_p, sh_d, i0w, i1w, r0a, r1a, r0b, r1b, out_v,
                   spa, sda, spb, sdb):
    wid = lax.axis_index("s") * NC + lax.axis_index("c")
    base_w = wid * E_PER_W
    sid = lax.axis_index("s")
    row0 = sid * (10000 // NS)
    pltpu.sync_copy(xp_hbm.at[pl.ds(row0, 10000 // NS)],
                    sh_p.at[pl.ds(row0, 10000 // NS)])
    pltpu.sync_copy(xd_hbm.at[pl.ds(row0, 10000 // NS)],
                    sh_d.at[pl.ds(row0, 10000 // NS)])
    plsc.subcore_barrier()
    pltpu.sync_copy(idx_hbm.at[0, pl.ds(base_w, E_PER_W)], i0w)
    pltpu.sync_copy(idx_hbm.at[1, pl.ds(base_w, E_PER_W)], i1w)

    lanes = lax.iota(jnp.int32, L)
    masks = [((lanes >> b) & 1) == 0 for b in range(4)]
    perms = [lanes ^ (1 << b) for b in range(4)]

    def issue(ci, r0, r1, sp, sd):
        off = ci * CHUNK
        pltpu.async_copy(sh_p.at[i0w.at[pl.ds(off, CHUNK)]], r0, sp)
        pltpu.async_copy(sh_d.at[i1w.at[pl.ds(off, CHUNK)]], r1, sd)

    def consume(ci, r0, r1, sp, sd):
        off = ci * CHUNK
        pltpu.make_async_copy(sh_p.at[i0w.at[pl.ds(off, CHUNK)]], r0,
                              sp).wait()
        pltpu.make_async_copy(sh_d.at[i1w.at[pl.ds(off, CHUNK)]], r1,
                              sd).wait()

        def group_body(g, c):
            accs = []
            for k in range(L):
                e = g * L + k
                p = []
                for j in range(4):
                    a = plsc.bitcast(r0[e, pl.ds(j * L, L)], jnp.bfloat16)
                    b = plsc.bitcast(r1[e, pl.ds(j * L, L)], jnp.bfloat16)
                    p.append(a * b)
                s = (p[0] + p[1]) + (p[2] + p[3])
                lo, hi = plsc.unpack(s, format=plsc.PackFormat.INTERLEAVED)
                accs.append(lo + hi)
            for b in range(4):
                m, pm = masks[b], perms[b]
                accs = [jnp.where(m, u, v) + _lane_take(jnp.where(m, v, u), pm)
                        for u, v in zip(accs[0::2], accs[1::2])]
            out_v[pl.ds(g * L, L)] = accs[0]
            return c

        lax.fori_loop(0, CHUNK // L, group_body, 0)
        base = base_w + ci * CHUNK
        pltpu.sync_copy(out_v, out_hbm.at[pl.ds(base, CHUNK)])

    bufa = (r0a, r1a, spa, sda)
    bufb = (r0b, r1b, spb, sdb)

    issue(0, *bufa)

    def pair_body(g, carry):
        ci = 2 * g
        issue(ci + 1, *bufb)
        consume(ci, *bufa)

        @pl.when(ci + 2 < NCHUNK)
        def _():
            issue(ci + 2, *bufa)

        consume(ci + 1, *bufb)
        return carry

    lax.fori_loop(0, NCHUNK // 2, pair_body, 0)
    consume(NCHUNK - 1, *bufa)


def _pack_bf16(x):
    u = lax.bitcast_convert_type(x.astype(jnp.bfloat16), jnp.uint16)
    u = u.astype(jnp.uint32)
    return lax.bitcast_convert_type(u[:, :W] | (u[:, W:] << 16), jnp.float32)


def kernel(x_patient, x_drug, edge_label_index):
    return _sc_dot_kernel(_pack_bf16(x_patient), _pack_bf16(x_drug),
                          edge_label_index)


# speedup vs baseline: 1.0555x; 1.0555x over previous
import functools

import jax
import jax.numpy as jnp
from jax import lax
from jax.experimental import pallas as pl
from jax.experimental.pallas import tpu as pltpu
from jax.experimental.pallas import tpu_sc as plsc

NE = 320000
D = 128
W = D // 2
NC, NS, L = 2, 16, 16
NW = NC * NS
E_PER_W = NE // NW
CHUNK = 400
NCHUNK = E_PER_W // CHUNK

_mesh = plsc.VectorSubcoreMesh(core_axis_name="c", subcore_axis_name="s")


def _lane_take(x, idx):
    dnums = lax.GatherDimensionNumbers(
        offset_dims=(), collapsed_slice_dims=(0,), start_index_map=(0,))
    return lax.gather(x, idx[:, None], dnums, slice_sizes=(1,),
                      mode=lax.GatherScatterMode.PROMISE_IN_BOUNDS)


@functools.partial(
    pl.kernel,
    mesh=_mesh,
    out_type=jax.ShapeDtypeStruct((NE,), jnp.float32),
    compiler_params=pltpu.CompilerParams(needs_layout_passes=False,
                                         use_tc_tiling_on_sc=False),
    scratch_types=[
        pltpu.VMEM((E_PER_W,), jnp.int32),
        pltpu.VMEM((E_PER_W,), jnp.int32),
        pltpu.VMEM((CHUNK, W), jnp.float32),
        pltpu.VMEM((CHUNK, W), jnp.float32),
        pltpu.VMEM((CHUNK, W), jnp.float32),
        pltpu.VMEM((CHUNK, W), jnp.float32),
        pltpu.VMEM((CHUNK,), jnp.float32),
        pltpu.SemaphoreType.DMA,
        pltpu.SemaphoreType.DMA,
        pltpu.SemaphoreType.DMA,
        pltpu.SemaphoreType.DMA,
    ],
)
def _sc_dot_kernel(xp_hbm, xd_hbm, idx_hbm, out_hbm,
                   i0w, i1w, r0a, r1a, r0b, r1b, out_v,
                   spa, sda, spb, sdb):
    wid = lax.axis_index("s") * NC + lax.axis_index("c")
    base_w = wid * E_PER_W
    pltpu.sync_copy(idx_hbm.at[pl.ds(base_w, E_PER_W)], i0w)
    pltpu.sync_copy(idx_hbm.at[pl.ds(NE + base_w, E_PER_W)], i1w)

    lanes = lax.iota(jnp.int32, L)
    masks = [((lanes >> b) & 1) == 0 for b in range(4)]
    perms = [lanes ^ (1 << b) for b in range(4)]

    def issue(ci, r0, r1, sp, sd):
        off = ci * CHUNK
        pltpu.async_copy(xp_hbm.at[i0w.at[pl.ds(off, CHUNK)]], r0, sp)
        pltpu.async_copy(xd_hbm.at[i1w.at[pl.ds(off, CHUNK)]], r1, sd)

    def consume(ci, r0, r1, sp, sd):
        off = ci * CHUNK
        pltpu.make_async_copy(xp_hbm.at[i0w.at[pl.ds(off, CHUNK)]], r0,
                              sp).wait()
        pltpu.make_async_copy(xd_hbm.at[i1w.at[pl.ds(off, CHUNK)]], r1,
                              sd).wait()

        def group_body(g, c):
            accs = []
            for k in range(L):
                e = g * L + k
                p = []
                for j in range(4):
                    a = plsc.bitcast(r0[e, pl.ds(j * L, L)], jnp.bfloat16)
                    b = plsc.bitcast(r1[e, pl.ds(j * L, L)], jnp.bfloat16)
                    p.append(a * b)
                s = (p[0] + p[1]) + (p[2] + p[3])
                lo, hi = plsc.unpack(s, format=plsc.PackFormat.INTERLEAVED)
                accs.append(lo + hi)
            for b in range(4):
                m, pm = masks[b], perms[b]
                accs = [jnp.where(m, u, v) + _lane_take(jnp.where(m, v, u), pm)
                        for u, v in zip(accs[0::2], accs[1::2])]
            out_v[pl.ds(g * L, L)] = accs[0]
            return c

        lax.fori_loop(0, CHUNK // L, group_body, 0)
        base = base_w + ci * CHUNK
        pltpu.sync_copy(out_v, out_hbm.at[pl.ds(base, CHUNK)])

    bufa = (r0a, r1a, spa, sda)
    bufb = (r0b, r1b, spb, sdb)

    issue(0, *bufa)

    def pair_body(g, carry):
        ci = 2 * g
        issue(ci + 1, *bufb)
        consume(ci, *bufa)

        @pl.when(ci + 2 < NCHUNK)
        def _():
            issue(ci + 2, *bufa)

        consume(ci + 1, *bufb)
        return carry

    lax.fori_loop(0, NCHUNK // 2, pair_body, 0)
    consume(NCHUNK - 1, *bufa)


def _pack_bf16(x):
    u = lax.bitcast_convert_type(x.astype(jnp.bfloat16), jnp.uint16)
    u = u.astype(jnp.uint32)
    return lax.bitcast_convert_type(u[:, :W] | (u[:, W:] << 16), jnp.float32)


def kernel(x_patient, x_drug, edge_label_index):
    return _sc_dot_kernel(_pack_bf16(x_patient), _pack_bf16(x_drug),
                          edge_label_index.reshape(-1))
